# baseline (speedup 1.0000x reference)
import functools
import math

import jax
import jax.numpy as jnp
from jax import lax
from jax.experimental import pallas as pl
from jax.experimental.pallas import tpu as pltpu
from jax.experimental.pallas import tpu_sc as plsc

D_MODEL = 32
SCALE = math.sqrt(float(D_MODEL))

NUM_CORES = 2
NUM_SUBCORES = 16
NUM_WORKERS = NUM_CORES * NUM_SUBCORES
LANES = 16

NRING = 4
LOOKAHEAD = 2


@functools.lru_cache(maxsize=None)
def _build(seq_len: int, batch: int):
    assert batch % NUM_WORKERS == 0
    bpw = batch // NUM_WORKERS
    ngroups = bpw // LANES

    mesh = plsc.VectorSubcoreMesh(core_axis_name="c", subcore_axis_name="s")

    @functools.partial(
        pl.kernel,
        mesh=mesh,
        out_type=jax.ShapeDtypeStruct((seq_len, D_MODEL, batch), jnp.float32),
        scratch_types=[pltpu.VMEM((bpw,), jnp.int32) for _ in range(NRING)]
        + [pltpu.VMEM((bpw, D_MODEL), jnp.float32) for _ in range(NRING)]
        + [pltpu.VMEM((D_MODEL, bpw), jnp.float32) for _ in range(NRING)]
        + [pltpu.SemaphoreType.DMA for _ in range(2 * NRING)],
        compiler_params=pltpu.CompilerParams(
            use_tc_tiling_on_sc=False, needs_layout_passes=False),
    )
    def gather_kernel(idx_hbm, table_hbm, out_hbm, *scratch):
        idxb = scratch[:NRING]
        rows = scratch[NRING:2 * NRING]
        plane = scratch[2 * NRING:3 * NRING]
        gsem = scratch[3 * NRING:3 * NRING + NRING]
        ssem = scratch[3 * NRING + NRING:]
        wid = lax.axis_index("s") * NUM_CORES + lax.axis_index("c")
        b0 = wid * bpw

        def fetch(s, slot):
            pltpu.sync_copy(idx_hbm.at[pl.ds(s * batch + b0, bpw)], idxb[slot])
            pltpu.async_copy(table_hbm.at[idxb[slot]], rows[slot], gsem[slot])

        def wait_gather(slot):
            pltpu.make_async_copy(
                table_hbm.at[idxb[slot]], rows[slot], gsem[slot]).wait()

        def store(s, slot):
            pltpu.async_copy(
                plane[slot],
                out_hbm.at[s, :, pl.ds(b0, bpw)],
                ssem[slot])

        def wait_store(s, slot):
            pltpu.make_async_copy(
                plane[slot],
                out_hbm.at[s, :, pl.ds(b0, bpw)],
                ssem[slot]).wait()

        def transpose_scale(slot):
            def per_d(d, carry):
                col = jnp.full((LANES,), d, dtype=jnp.int32)
                for g in range(ngroups):
                    ridx = lax.iota(jnp.int32, LANES) + (g * LANES)
                    v = plsc.load_gather(rows[slot], [ridx, col])
                    plane[slot][d, g * LANES:(g + 1) * LANES] = v * SCALE
                return carry

            lax.fori_loop(0, D_MODEL, per_d, 0)

        for s in range(LOOKAHEAD):
            fetch(s, s % NRING)

        def step(s, slot):
            wait_gather(slot)
            transpose_scale(slot)
            store(s, slot)

        for s in range(LOOKAHEAD):
            step(s, s % NRING)
            fetch(s + LOOKAHEAD, (s + LOOKAHEAD) % NRING)

        n_steady = seq_len - 2 * LOOKAHEAD
        assert n_steady % NRING == 0

        def steady(so, carry):
            for u in range(NRING):
                s = LOOKAHEAD + so * NRING + u
                slot = (LOOKAHEAD + u) % NRING
                step(s, slot)
                wait_store(s - LOOKAHEAD, (slot + LOOKAHEAD) % NRING)
                fetch(s + LOOKAHEAD, (slot + LOOKAHEAD) % NRING)
            return carry

        lax.fori_loop(0, n_steady // NRING, steady, 0)

        for s in range(seq_len - LOOKAHEAD, seq_len):
            step(s, s % NRING)

        for s in range(seq_len - NRING, seq_len):
            wait_store(s, s % NRING)

    return gather_kernel


def kernel(inputs, embedding_matrix):
    b, s = inputs.shape
    idx = inputs.T.reshape(s * b).astype(jnp.int32)
    out_t = _build(s, b)(idx, embedding_matrix)
    return jnp.transpose(out_t, (2, 0, 1))


# speedup vs baseline: 1.0721x; 1.0721x over previous
import functools
import math

import jax
import jax.numpy as jnp
from jax import lax
from jax.experimental import pallas as pl
from jax.experimental.pallas import tpu as pltpu
from jax.experimental.pallas import tpu_sc as plsc

D_MODEL = 32
SCALE = math.sqrt(float(D_MODEL))

NUM_CORES = 2
NUM_SUBCORES = 16
NUM_WORKERS = NUM_CORES * NUM_SUBCORES
LANES = 16

S_CHUNK = 4
NRING = 2


@functools.lru_cache(maxsize=None)
def _build(seq_len: int, batch: int):
    assert batch % NUM_WORKERS == 0
    bpw = batch // NUM_WORKERS
    rows_per_step = S_CHUNK * bpw
    nsteps = seq_len // S_CHUNK
    assert seq_len % S_CHUNK == 0
    nslices = rows_per_step // LANES

    mesh = plsc.VectorSubcoreMesh(core_axis_name="c", subcore_axis_name="s")

    @functools.partial(
        pl.kernel,
        mesh=mesh,
        out_type=jax.ShapeDtypeStruct((seq_len, D_MODEL, batch), jnp.float32),
        scratch_types=[pltpu.VMEM((seq_len, bpw), jnp.int32)]
        + [pltpu.VMEM((rows_per_step, D_MODEL), jnp.float32) for _ in range(NRING)]
        + [pltpu.VMEM((S_CHUNK, D_MODEL, bpw), jnp.float32) for _ in range(NRING)]
        + [pltpu.SemaphoreType.DMA for _ in range(2 * NRING)],
        compiler_params=pltpu.CompilerParams(
            use_tc_tiling_on_sc=False, needs_layout_passes=False),
    )
    def gather_kernel(idx_hbm, table_hbm, out_hbm, idx_all, *scratch):
        rows = scratch[:NRING]
        plane = scratch[NRING:2 * NRING]
        gsem = scratch[2 * NRING:3 * NRING]
        ssem = scratch[3 * NRING:]
        wid = lax.axis_index("s") * NUM_CORES + lax.axis_index("c")
        b0 = wid * bpw

        pltpu.sync_copy(idx_hbm.at[:, pl.ds(b0, bpw)], idx_all)

        def fetch(i, slot):
            for k in range(S_CHUNK):
                pltpu.async_copy(
                    table_hbm.at[idx_all.at[i * S_CHUNK + k]],
                    rows[slot].at[pl.ds(k * bpw, bpw)], gsem[slot])

        def wait_gather(i, slot):
            for k in range(S_CHUNK):
                pltpu.make_async_copy(
                    table_hbm.at[idx_all.at[i * S_CHUNK + k]],
                    rows[slot].at[pl.ds(k * bpw, bpw)], gsem[slot]).wait()

        def store(i, slot):
            pltpu.async_copy(
                plane[slot],
                out_hbm.at[pl.ds(i * S_CHUNK, S_CHUNK), :, pl.ds(b0, bpw)],
                ssem[slot])

        def wait_store(i, slot):
            pltpu.make_async_copy(
                plane[slot],
                out_hbm.at[pl.ds(i * S_CHUNK, S_CHUNK), :, pl.ds(b0, bpw)],
                ssem[slot]).wait()

        def transpose_scale(slot):
            def body(i, carry):
                ridx = lax.iota(jnp.int32, LANES) + i * LANES
                s_local = i // (bpw // LANES)
                g16 = (i % (bpw // LANES)) * LANES
                for d in range(D_MODEL):
                    col = jnp.full((LANES,), d, dtype=jnp.int32)
                    v = plsc.load_gather(rows[slot], [ridx, col])
                    plane[slot][s_local, d, pl.ds(g16, LANES)] = v * SCALE
                return carry

            lax.fori_loop(0, nslices, body, 0)

        def step(i, slot, head=False, tail=False):
            wait_gather(i, slot)
            if not head:
                wait_store(i - NRING, slot)
            transpose_scale(slot)
            store(i, slot)
            if not tail:
                fetch(i + NRING, slot)

        fetch(0, 0)
        fetch(1, 1)
        step(0, 0, head=True)
        step(1, 1, head=True)

        n_steady = nsteps - 2 * NRING
        assert n_steady % 2 == 0

        def steady(p, carry):
            i = NRING + p * 2
            step(i, 0)
            step(i + 1, 1)
            return carry

        lax.fori_loop(0, n_steady // 2, steady, 0)

        step(nsteps - 2, 0, tail=True)
        step(nsteps - 1, 1, tail=True)

        wait_store(nsteps - 2, 0)
        wait_store(nsteps - 1, 1)

    return gather_kernel


def kernel(inputs, embedding_matrix):
    b, s = inputs.shape
    idx = inputs.T.astype(jnp.int32)
    out_t = _build(s, b)(idx, embedding_matrix)
    return jnp.transpose(out_t, (2, 0, 1))


# speedup vs baseline: 2.0168x; 1.8812x over previous
import functools
import math

import jax
import jax.numpy as jnp
from jax import lax
from jax.experimental import pallas as pl
from jax.experimental.pallas import tpu as pltpu
from jax.experimental.pallas import tpu_sc as plsc

D_MODEL = 32
SCALE = math.sqrt(float(D_MODEL))

NUM_CORES = 2
NUM_SUBCORES = 16
NUM_WORKERS = NUM_CORES * NUM_SUBCORES
LANES = 16

S_CHUNK = 4
NRING = 2


@functools.lru_cache(maxsize=None)
def _build(seq_len: int, batch: int):
    assert batch % NUM_WORKERS == 0
    bpw = batch // NUM_WORKERS
    rows_per_step = S_CHUNK * bpw
    nsteps = seq_len // S_CHUNK
    assert seq_len % S_CHUNK == 0
    nslices = rows_per_step // LANES

    mesh = plsc.VectorSubcoreMesh(core_axis_name="c", subcore_axis_name="s")

    @functools.partial(
        pl.kernel,
        mesh=mesh,
        out_type=jax.ShapeDtypeStruct((seq_len, D_MODEL, batch), jnp.float32),
        scratch_types=[pltpu.VMEM((seq_len, bpw), jnp.int32)]
        + [pltpu.VMEM((rows_per_step, D_MODEL), jnp.float32) for _ in range(NRING)]
        + [pltpu.VMEM((S_CHUNK * D_MODEL, bpw), jnp.float32) for _ in range(NRING)]
        + [pltpu.SemaphoreType.DMA for _ in range(2 * NRING)],
        compiler_params=pltpu.CompilerParams(
            use_tc_tiling_on_sc=False, needs_layout_passes=False),
    )
    def gather_kernel(idx_hbm, table_hbm, out_hbm, idx_all, *scratch):
        rows = scratch[:NRING]
        plane = scratch[NRING:2 * NRING]
        gsem = scratch[2 * NRING:3 * NRING]
        ssem = scratch[3 * NRING:]
        wid = lax.axis_index("s") * NUM_CORES + lax.axis_index("c")
        b0 = wid * bpw

        pltpu.sync_copy(idx_hbm.at[:, pl.ds(b0, bpw)], idx_all)

        def fetch(i, slot):
            for k in range(S_CHUNK):
                pltpu.async_copy(
                    table_hbm.at[idx_all.at[i * S_CHUNK + k]],
                    rows[slot].at[pl.ds(k * bpw, bpw)], gsem[slot])

        def wait_gather(i, slot):
            for k in range(S_CHUNK):
                pltpu.make_async_copy(
                    table_hbm.at[idx_all.at[i * S_CHUNK + k]],
                    rows[slot].at[pl.ds(k * bpw, bpw)], gsem[slot]).wait()

        def store(i, slot):
            for k in range(S_CHUNK):
                pltpu.async_copy(
                    plane[slot].at[pl.ds(k * D_MODEL, D_MODEL)],
                    out_hbm.at[i * S_CHUNK + k, :, pl.ds(b0, bpw)],
                    ssem[slot])

        def wait_store(i, slot):
            for k in range(S_CHUNK):
                pltpu.make_async_copy(
                    plane[slot].at[pl.ds(k * D_MODEL, D_MODEL)],
                    out_hbm.at[i * S_CHUNK + k, :, pl.ds(b0, bpw)],
                    ssem[slot]).wait()

        lane_iota = lax.iota(jnp.int32, LANES)
        cols = [jnp.full((LANES,), d, dtype=jnp.int32) for d in range(D_MODEL)]
        groups_per_plane = bpw // LANES

        def transpose_scale(slot):
            @functools.partial(plsc.parallel_loop, 0, nslices, unroll=2)
            def body(i):
                ridx = lane_iota + i * LANES
                row0 = (i // groups_per_plane) * D_MODEL
                g16 = (i % groups_per_plane) * LANES
                for d in range(D_MODEL):
                    v = plsc.load_gather(rows[slot], [ridx, cols[d]])
                    plane[slot][row0 + d, pl.ds(g16, LANES)] = v * SCALE

        def step(i, slot, head=False, tail=False):
            wait_gather(i, slot)
            if not head:
                wait_store(i - NRING, slot)
            transpose_scale(slot)
            store(i, slot)
            if not tail:
                fetch(i + NRING, slot)

        fetch(0, 0)
        fetch(1, 1)
        step(0, 0, head=True)
        step(1, 1, head=True)

        n_steady = nsteps - 2 * NRING
        assert n_steady % 2 == 0

        def steady(p, carry):
            i = NRING + p * 2
            step(i, 0)
            step(i + 1, 1)
            return carry

        lax.fori_loop(0, n_steady // 2, steady, 0)

        step(nsteps - 2, 0, tail=True)
        step(nsteps - 1, 1, tail=True)

        wait_store(nsteps - 2, 0)
        wait_store(nsteps - 1, 1)

    return gather_kernel


def kernel(inputs, embedding_matrix):
    b, s = inputs.shape
    idx = inputs.T.astype(jnp.int32)
    out_t = _build(s, b)(idx, embedding_matrix)
    return jnp.transpose(out_t, (2, 0, 1))
